# Initial kernel scaffold; baseline (speedup 1.0000x reference)
import jax
import jax.numpy as jnp
from jax.experimental import pallas as pl


def kernel(x, edge_index, W0, b0, W1, b1, W2, b2):
    raise NotImplementedError("write your pallas kernel here")


# speedup vs baseline: 5.6222x; 5.6222x over previous
import functools

import jax
import jax.numpy as jnp
from jax import lax
from jax.experimental import pallas as pl
from jax.experimental.pallas import tpu as pltpu
from jax.experimental.pallas import tpu_sc as plsc

N = 10000
E = 320000
BLK = 128
NB = E // BLK
NCORE = 2
NSUB = 16
RPT = 624
TAIL = N - NSUB * RPT
ZR = 208
DEGW = 16


def _core_sub():
    return lax.axis_index("c"), lax.axis_index("s")


def _make_deg_kernel():
    mesh = plsc.VectorSubcoreMesh(core_axis_name="c", subcore_axis_name="s")
    nblk_core = NB // NCORE

    @functools.partial(
        pl.kernel,
        out_type=(
            jax.ShapeDtypeStruct((NCORE, N, DEGW), jnp.float32),
            jax.ShapeDtypeStruct((NCORE, N, DEGW), jnp.float32),
        ),
        mesh=mesh,
        scratch_types=[
            pltpu.VMEM_SHARED((N, DEGW), jnp.float32),
            pltpu.VMEM_SHARED((N, DEGW), jnp.float32),
            pltpu.VMEM((BLK,), jnp.int32),
            pltpu.VMEM((BLK,), jnp.int32),
            pltpu.VMEM((BLK, DEGW), jnp.float32),
        ],
        compiler_params=pltpu.CompilerParams(use_tc_tiling_on_sc=False),
    )
    def deg_kernel(src_hbm, dst_hbm, ones_hbm, zeros_hbm,
                   outs_hbm, outd_hbm, acc_s, acc_d, src_v, dst_v, ones_v):
        c, s = _core_sub()
        r0 = s * RPT
        pltpu.sync_copy(ones_hbm, ones_v)
        for z in range(RPT // ZR):
            pltpu.sync_copy(zeros_hbm.at[pl.ds(0, ZR)],
                            acc_s.at[pl.ds(r0 + z * ZR, ZR)])
            pltpu.sync_copy(zeros_hbm.at[pl.ds(0, ZR)],
                            acc_d.at[pl.ds(r0 + z * ZR, ZR)])

        @pl.when(s == NSUB - 1)
        def _():
            pltpu.sync_copy(zeros_hbm.at[pl.ds(0, TAIL)],
                            acc_s.at[pl.ds(NSUB * RPT, TAIL)])
            pltpu.sync_copy(zeros_hbm.at[pl.ds(0, TAIL)],
                            acc_d.at[pl.ds(NSUB * RPT, TAIL)])

        plsc.subcore_barrier()

        base = c * nblk_core
        nblk = (nblk_core - s + NSUB - 1) // NSUB

        def body(i, carry):
            blk = base + s + i * NSUB
            pltpu.sync_copy(src_hbm.at[blk], src_v)
            pltpu.sync_copy(dst_hbm.at[blk], dst_v)
            pltpu.sync_copy(ones_v, acc_s.at[src_v], add=True)
            pltpu.sync_copy(ones_v, acc_d.at[dst_v], add=True)
            return carry

        lax.fori_loop(0, nblk, body, 0)
        plsc.subcore_barrier()
        pltpu.sync_copy(acc_s.at[pl.ds(r0, RPT)], outs_hbm.at[c, pl.ds(r0, RPT)])
        pltpu.sync_copy(acc_d.at[pl.ds(r0, RPT)], outd_hbm.at[c, pl.ds(r0, RPT)])

        @pl.when(s == NSUB - 1)
        def _():
            t0 = NSUB * RPT
            pltpu.sync_copy(acc_s.at[pl.ds(t0, TAIL)],
                            outs_hbm.at[c, pl.ds(t0, TAIL)])
            pltpu.sync_copy(acc_d.at[pl.ds(t0, TAIL)],
                            outd_hbm.at[c, pl.ds(t0, TAIL)])

    return deg_kernel


def _make_agg_kernel(D):
    mesh = plsc.VectorSubcoreMesh(core_axis_name="c", subcore_axis_name="s")
    nblk_core = NB // NCORE

    @functools.partial(
        pl.kernel,
        out_type=jax.ShapeDtypeStruct((NCORE, N, D), jnp.float32),
        mesh=mesh,
        scratch_types=[
            pltpu.VMEM_SHARED((N, D), jnp.float32),
            pltpu.VMEM((BLK,), jnp.int32),
            pltpu.VMEM((BLK,), jnp.int32),
            pltpu.VMEM((BLK, D), jnp.float32),
            pltpu.SemaphoreType.DMA,
        ],
        compiler_params=pltpu.CompilerParams(use_tc_tiling_on_sc=False),
    )
    def agg_kernel(y_hbm, src_hbm, dst_hbm, zeros_hbm,
                   out_hbm, acc, src_v, dst_v, rows_v, sem):
        c, s = _core_sub()
        r0 = s * RPT
        for z in range(RPT // ZR):
            pltpu.sync_copy(zeros_hbm, acc.at[pl.ds(r0 + z * ZR, ZR)])

        @pl.when(s == NSUB - 1)
        def _():
            pltpu.sync_copy(zeros_hbm.at[pl.ds(0, TAIL)],
                            acc.at[pl.ds(NSUB * RPT, TAIL)])

        plsc.subcore_barrier()

        base = c * nblk_core
        nblk = (nblk_core - s + NSUB - 1) // NSUB

        def body(i, carry):
            blk = base + s + i * NSUB
            pltpu.sync_copy(src_hbm.at[blk], src_v)
            pltpu.sync_copy(dst_hbm.at[blk], dst_v)
            pltpu.async_copy(y_hbm.at[src_v], rows_v, sem).wait()
            pltpu.sync_copy(rows_v, acc.at[dst_v], add=True)
            return carry

        lax.fori_loop(0, nblk, body, 0)
        plsc.subcore_barrier()
        pltpu.sync_copy(acc.at[pl.ds(r0, RPT)], out_hbm.at[c, pl.ds(r0, RPT)])

        @pl.when(s == NSUB - 1)
        def _():
            t0 = NSUB * RPT
            pltpu.sync_copy(acc.at[pl.ds(t0, TAIL)],
                            out_hbm.at[c, pl.ds(t0, TAIL)])

    return agg_kernel


_deg_kernel = _make_deg_kernel()
_agg128 = _make_agg_kernel(128)
_agg64 = _make_agg_kernel(64)


_RB = 1000
_GRID = N // _RB


def _tc0_body(hs_ref, hd_ref, x_ref, ns_ref, nd_ref, y0_ref):
    ds = jnp.sum(hs_ref[...], axis=(0, 2)) * (1.0 / DEGW)
    dd = jnp.sum(hd_ref[...], axis=(0, 2)) * (1.0 / DEGW)
    ns = lax.rsqrt(jnp.maximum(ds, 1.0))
    nd = lax.rsqrt(jnp.maximum(dd, 1.0))
    ns_ref[...] = ns[:, None]
    nd_ref[...] = nd[:, None]
    y0_ref[...] = x_ref[...] * ns[:, None]


def _tc0(hs, hd, x):
    return pl.pallas_call(
        _tc0_body,
        grid=(_GRID,),
        in_specs=[
            pl.BlockSpec((NCORE, _RB, DEGW), lambda i: (0, i, 0)),
            pl.BlockSpec((NCORE, _RB, DEGW), lambda i: (0, i, 0)),
            pl.BlockSpec((_RB, 128), lambda i: (i, 0)),
        ],
        out_specs=[
            pl.BlockSpec((_RB, 1), lambda i: (i, 0)),
            pl.BlockSpec((_RB, 1), lambda i: (i, 0)),
            pl.BlockSpec((_RB, 128), lambda i: (i, 0)),
        ],
        out_shape=[
            jax.ShapeDtypeStruct((N, 1), jnp.float32),
            jax.ShapeDtypeStruct((N, 1), jnp.float32),
            jax.ShapeDtypeStruct((N, 128), jnp.float32),
        ],
    )(hs, hd, x)


def _tc1_body(g0_ref, ns_ref, nd_ref, w0_ref, b0_ref, y1a_ref, y1b_ref):
    g0 = (g0_ref[0] + g0_ref[1]) * nd_ref[...]
    h = jnp.dot(g0, w0_ref[...], preferred_element_type=jnp.float32)
    h = jnp.maximum(h + b0_ref[...], 0.0) * ns_ref[...]
    y1a_ref[...] = h[:, :128]
    y1b_ref[...] = h[:, 128:]


def _tc1(g0, ns, nd, W0, b0):
    return pl.pallas_call(
        _tc1_body,
        grid=(_GRID,),
        in_specs=[
            pl.BlockSpec((NCORE, _RB, 128), lambda i: (0, i, 0)),
            pl.BlockSpec((_RB, 1), lambda i: (i, 0)),
            pl.BlockSpec((_RB, 1), lambda i: (i, 0)),
            pl.BlockSpec((128, 256), lambda i: (0, 0)),
            pl.BlockSpec((1, 256), lambda i: (0, 0)),
        ],
        out_specs=[
            pl.BlockSpec((_RB, 128), lambda i: (i, 0)),
            pl.BlockSpec((_RB, 128), lambda i: (i, 0)),
        ],
        out_shape=[
            jax.ShapeDtypeStruct((N, 128), jnp.float32),
            jax.ShapeDtypeStruct((N, 128), jnp.float32),
        ],
    )(g0, ns, nd, W0, b0)


def _tc2_body(g1a_ref, g1b_ref, ns_ref, nd_ref, w1_ref, b1_ref, w2_ref, y2_ref):
    a = (g1a_ref[0] + g1a_ref[1]) * nd_ref[...]
    b = (g1b_ref[0] + g1b_ref[1]) * nd_ref[...]
    h = jnp.dot(a, w1_ref[0], preferred_element_type=jnp.float32)
    h = h + jnp.dot(b, w1_ref[1], preferred_element_type=jnp.float32)
    h = jnp.maximum(h + b1_ref[...], 0.0)
    t = jnp.dot(h, w2_ref[...], preferred_element_type=jnp.float32)
    y2_ref[...] = t * ns_ref[...]


def _tc2(g1a, g1b, ns, nd, W1, b1, W2):
    return pl.pallas_call(
        _tc2_body,
        grid=(_GRID,),
        in_specs=[
            pl.BlockSpec((NCORE, _RB, 128), lambda i: (0, i, 0)),
            pl.BlockSpec((NCORE, _RB, 128), lambda i: (0, i, 0)),
            pl.BlockSpec((_RB, 1), lambda i: (i, 0)),
            pl.BlockSpec((_RB, 1), lambda i: (i, 0)),
            pl.BlockSpec((NCORE, 128, 256), lambda i: (0, 0, 0)),
            pl.BlockSpec((1, 256), lambda i: (0, 0)),
            pl.BlockSpec((256, 64), lambda i: (0, 0)),
        ],
        out_specs=pl.BlockSpec((_RB, 64), lambda i: (i, 0)),
        out_shape=jax.ShapeDtypeStruct((N, 64), jnp.float32),
    )(g1a, g1b, ns, nd, W1, b1, W2)


def _tc3_body(g2_ref, nd_ref, b2_ref, out_ref):
    out_ref[...] = (g2_ref[0] + g2_ref[1]) * nd_ref[...] + b2_ref[...]


def _tc3(g2, nd, b2):
    return pl.pallas_call(
        _tc3_body,
        grid=(_GRID,),
        in_specs=[
            pl.BlockSpec((NCORE, _RB, 64), lambda i: (0, i, 0)),
            pl.BlockSpec((_RB, 1), lambda i: (i, 0)),
            pl.BlockSpec((1, 64), lambda i: (0, 0)),
        ],
        out_specs=pl.BlockSpec((_RB, 64), lambda i: (i, 0)),
        out_shape=jax.ShapeDtypeStruct((N, 64), jnp.float32),
    )(g2, nd, b2)


@jax.jit
def _run(x, edge_index, W0, b0, W1, b1, W2, b2):
    src2d = edge_index[0].astype(jnp.int32).reshape(NB, BLK)
    dst2d = edge_index[1].astype(jnp.int32).reshape(NB, BLK)
    ones = jnp.ones((BLK, DEGW), jnp.float32)
    zeros_deg = jnp.zeros((ZR, DEGW), jnp.float32)
    zeros128 = jnp.zeros((ZR, 128), jnp.float32)
    zeros64 = jnp.zeros((ZR, 64), jnp.float32)

    hs, hd = _deg_kernel(src2d, dst2d, ones, zeros_deg)
    ns, nd, y0 = _tc0(hs, hd, x)
    g0 = _agg128(y0, src2d, dst2d, zeros128)
    y1a, y1b = _tc1(g0, ns, nd, W0, b0.reshape(1, -1))
    g1a = _agg128(y1a, src2d, dst2d, zeros128)
    g1b = _agg128(y1b, src2d, dst2d, zeros128)
    y2 = _tc2(g1a, g1b, ns, nd, W1.reshape(NCORE, 128, 256), b1.reshape(1, -1), W2)
    g2 = _agg64(y2, src2d, dst2d, zeros64)
    return _tc3(g2, nd, b2.reshape(1, -1))


def kernel(x, edge_index, W0, b0, W1, b1, W2, b2):
    return _run(x, edge_index, W0, b0, W1, b1, W2, b2)
